# baseline (speedup 1.0000x reference)
import functools

import jax
import jax.numpy as jnp
from jax import lax
from jax.experimental import pallas as pl
from jax.experimental.pallas import tpu as pltpu
from jax.experimental.pallas import tpu_sc as plsc

VOCAB = 100000
EMB_DIM = 1024
BATCH = 4096
BAG = 20

NUM_CORES = 2
NUM_SUBCORES = 16
NUM_WORKERS = NUM_CORES * NUM_SUBCORES
ROWS_PER_WORKER = BATCH // NUM_WORKERS

TAIL = 32000
SPLIT = VOCAB - TAIL
ROWS_PER_STEP = 2000
SC_ROWS_PER_SUB = TAIL // NUM_SUBCORES
SC_CHUNK = 80
LANE_CHUNKS = EMB_DIM // 16


def _head_body(w1aT_ref, w1bT_ref, w2aT_ref, w2bT_ref,
               b1a_ref, b1b_ref, b2a_ref, b2b_ref, v1_ref, v12_ref, off_ref):
    v1 = jnp.dot(w1bT_ref[...], w1aT_ref[...], preferred_element_type=jnp.float32)
    v2 = jnp.dot(w2bT_ref[...], w2aT_ref[...], preferred_element_type=jnp.float32)
    c = (jnp.sum(b1a_ref[...] * w1bT_ref[...])
         + jnp.sum(b2a_ref[...] * w2bT_ref[...]))
    v1_ref[...] = v1
    v12_ref[...] = v1 + v2
    off_ref[...] = jnp.full((1, 1), (c + b1b_ref[0, 0] + b2b_ref[0, 0])
                            * (1.0 / (2.0 * BAG)), jnp.float32)


def _head_vectors(w1a, w1b, w2a, w2b, b1a, b1b, b2a, b2b):
    return pl.pallas_call(
        _head_body,
        out_shape=[
            jax.ShapeDtypeStruct((1, EMB_DIM), jnp.float32),
            jax.ShapeDtypeStruct((1, EMB_DIM), jnp.float32),
            jax.ShapeDtypeStruct((1, 1), jnp.float32),
        ],
    )(w1a.T, w1b.T, w2a.T, w2b.T,
      b1a.reshape(1, 512), b1b.reshape(1, 1),
      b2a.reshape(1, 512), b2b.reshape(1, 1))


def _project_body(t1_ref, t2_ref, v1_ref, v12_ref, off_ref, s1_ref, s2_ref):
    off = off_ref[0, 0]
    s1_ref[...] = jnp.sum(t1_ref[...] * v1_ref[...], axis=1, keepdims=True) + off
    s2_ref[...] = jnp.sum(t2_ref[...] * v12_ref[...], axis=1, keepdims=True) + off


def _project_tables_tc(table_1, table_2, v1, v12, off):
    grid = (SPLIT // ROWS_PER_STEP,)
    s1, s2 = pl.pallas_call(
        _project_body,
        grid=grid,
        in_specs=[
            pl.BlockSpec((ROWS_PER_STEP, EMB_DIM), lambda i: (i, 0)),
            pl.BlockSpec((ROWS_PER_STEP, EMB_DIM), lambda i: (i, 0)),
            pl.BlockSpec((1, EMB_DIM), lambda i: (0, 0)),
            pl.BlockSpec((1, EMB_DIM), lambda i: (0, 0)),
            pl.BlockSpec((1, 1), lambda i: (0, 0)),
        ],
        out_specs=[
            pl.BlockSpec((ROWS_PER_STEP, 1), lambda i: (i, 0)),
            pl.BlockSpec((ROWS_PER_STEP, 1), lambda i: (i, 0)),
        ],
        out_shape=[
            jax.ShapeDtypeStruct((SPLIT, 1), jnp.float32),
            jax.ShapeDtypeStruct((SPLIT, 1), jnp.float32),
        ],
    )(table_1, table_2, v1, v12, off)
    return s1.reshape(SPLIT), s2.reshape(SPLIT)


def _sc_project_one(t_hbm, v_hbm, sub, tail_hbm, buf_v, v_v, part_v, sem):
    pltpu.sync_copy(v_hbm, v_v)
    base = SPLIT + sub * SC_ROWS_PER_SUB

    def chunk_body(ci, _):
        pltpu.async_copy(
            t_hbm.at[pl.ds(base + ci * SC_CHUNK, SC_CHUNK)], buf_v, sem).wait()
        for g in range(SC_CHUNK // 16):
            zeros = jnp.zeros((16,), jnp.float32)

            def col_body(k, accs):
                ks = pl.multiple_of(k * 16, 16)
                vk = v_v[pl.ds(ks, 16)]
                return tuple(accs[r] + buf_v[g * 16 + r, pl.ds(ks, 16)] * vk
                             for r in range(16))

            accs = lax.fori_loop(0, LANE_CHUNKS, col_body, (zeros,) * 16)
            for r in range(16):
                part_v[g * 16 + r, pl.ds(0, 16)] = accs[r]
        pltpu.sync_copy(
            part_v,
            tail_hbm.at[pl.ds(sub * SC_ROWS_PER_SUB + ci * SC_CHUNK, SC_CHUNK)])
        return 0

    lax.fori_loop(0, SC_ROWS_PER_SUB // SC_CHUNK, chunk_body, 0)


def _sc_project_body(t1_hbm, t2_hbm, v1_hbm, v12_hbm,
                     tail1_hbm, tail2_hbm, buf_v, v_v, part_v, sem):
    core = lax.axis_index("c")
    sub = lax.axis_index("s")

    @pl.when(core == 0)
    def _():
        _sc_project_one(t1_hbm, v1_hbm, sub, tail1_hbm,
                        buf_v, v_v, part_v, sem)

    @pl.when(core == 1)
    def _():
        _sc_project_one(t2_hbm, v12_hbm, sub, tail2_hbm,
                        buf_v, v_v, part_v, sem)


@functools.lru_cache(maxsize=None)
def _sc_project_tails():
    return functools.partial(
        pl.kernel,
        mesh=plsc.VectorSubcoreMesh(core_axis_name="c", subcore_axis_name="s"),
        out_type=[jax.ShapeDtypeStruct((TAIL, 16), jnp.float32),
                  jax.ShapeDtypeStruct((TAIL, 16), jnp.float32)],
        scratch_types=[
            pltpu.VMEM((SC_CHUNK, EMB_DIM), jnp.float32),
            pltpu.VMEM((EMB_DIM,), jnp.float32),
            pltpu.VMEM((SC_CHUNK, 16), jnp.float32),
            pltpu.SemaphoreType.DMA,
        ],
    )(_sc_project_body)


def _tail_reduce_body(p1_ref, p2_ref, off_ref, s1_ref, s2_ref):
    off = off_ref[0, 0]
    grp = lax.broadcasted_iota(jnp.int32, (128, 8), 0) // 16
    col = lax.broadcasted_iota(jnp.int32, (128, 8), 1)
    m = (grp == col).astype(jnp.float32)
    s1_ref[...] = jnp.dot(p1_ref[...], m, preferred_element_type=jnp.float32) + off
    s2_ref[...] = jnp.dot(p2_ref[...], m, preferred_element_type=jnp.float32) + off


def _tail_reduce(p1, p2, off):
    s1, s2 = pl.pallas_call(
        _tail_reduce_body,
        out_shape=[
            jax.ShapeDtypeStruct((TAIL // 8, 8), jnp.float32),
            jax.ShapeDtypeStruct((TAIL // 8, 8), jnp.float32),
        ],
    )(p1.reshape(TAIL // 8, 128), p2.reshape(TAIL // 8, 128), off)
    return s1.reshape(TAIL), s2.reshape(TAIL)


def _sc_body(s1_hbm, s2_hbm, xt_hbm, yt_hbm, out_hbm,
             idx1_v, idx2_v, vals1_v, vals2_v, out_v, sem):
    wid = lax.axis_index("s") * NUM_CORES + lax.axis_index("c")
    base = wid * ROWS_PER_WORKER

    pltpu.sync_copy(xt_hbm.at[wid], idx1_v)
    pltpu.sync_copy(yt_hbm.at[wid], idx2_v)
    descs = []
    for j in range(BAG):
        descs.append(pltpu.async_copy(s1_hbm.at[idx1_v.at[j]], vals1_v.at[j], sem))
        descs.append(pltpu.async_copy(s2_hbm.at[idx2_v.at[j]], vals2_v.at[j], sem))
    for d in descs:
        d.wait()

    for k in range(ROWS_PER_WORKER // 16):
        sl = pl.ds(k * 16, 16)
        acc = vals1_v[0, sl] + vals2_v[0, sl]
        for j in range(1, BAG):
            acc = acc + vals1_v[j, sl]
            acc = acc + vals2_v[j, sl]
        out_v[sl] = acc

    pltpu.sync_copy(out_v, out_hbm.at[pl.ds(base, ROWS_PER_WORKER)])


@functools.lru_cache(maxsize=None)
def _sc_bag_sum():
    return functools.partial(
        pl.kernel,
        mesh=plsc.VectorSubcoreMesh(core_axis_name="c", subcore_axis_name="s"),
        out_type=jax.ShapeDtypeStruct((BATCH,), jnp.float32),
        scratch_types=[
            pltpu.VMEM((BAG, ROWS_PER_WORKER), jnp.int32),
            pltpu.VMEM((BAG, ROWS_PER_WORKER), jnp.int32),
            pltpu.VMEM((BAG, ROWS_PER_WORKER), jnp.float32),
            pltpu.VMEM((BAG, ROWS_PER_WORKER), jnp.float32),
            pltpu.VMEM((ROWS_PER_WORKER,), jnp.float32),
            pltpu.SemaphoreType.DMA,
        ],
    )(_sc_body)


def kernel(x, y, table_1, table_2, w1a, b1a, w1b, b1b, w2a, b2a, w2b, b2b):
    v1, v12, off = _head_vectors(w1a, w1b, w2a, w2b, b1a, b1b, b2a, b2b)
    s1_head, s2_head = _project_tables_tc(table_1, table_2, v1, v12, off)
    p1, p2 = _sc_project_tails()(
        table_1, table_2, v1.reshape(EMB_DIM), v12.reshape(EMB_DIM))
    tail1, tail2 = _tail_reduce(p1, p2, off)
    s1 = jnp.concatenate([s1_head, tail1])
    s2 = jnp.concatenate([s2_head, tail2])
    xt = x.reshape(NUM_WORKERS, ROWS_PER_WORKER, BAG).transpose(0, 2, 1)
    yt = y.reshape(NUM_WORKERS, ROWS_PER_WORKER, BAG).transpose(0, 2, 1)
    out = _sc_bag_sum()(s1, s2, xt, yt)
    return out.reshape(BATCH, 1)


# speedup vs baseline: 1.0942x; 1.0942x over previous
import functools

import jax
import jax.numpy as jnp
from jax import lax
from jax.experimental import pallas as pl
from jax.experimental.pallas import tpu as pltpu
from jax.experimental.pallas import tpu_sc as plsc

VOCAB = 100000
EMB_DIM = 1024
BATCH = 4096
BAG = 20


ROWS_PER_STEP = 2000


def _project_body(t1_ref, t2_ref, w1aT_ref, w1bT_ref, w2aT_ref, w2bT_ref,
                  b1a_ref, b1b_ref, b2a_ref, b2b_ref, s1_ref, s2_ref):
    v1 = jnp.dot(w1bT_ref[...], w1aT_ref[...], preferred_element_type=jnp.float32)
    v2 = jnp.dot(w2bT_ref[...], w2aT_ref[...], preferred_element_type=jnp.float32)
    c = (jnp.sum(b1a_ref[...] * w1bT_ref[...])
         + jnp.sum(b2a_ref[...] * w2bT_ref[...]))
    off = (c + b1b_ref[0, 0] + b2b_ref[0, 0]) * (1.0 / (2.0 * BAG))
    s1_ref[...] = jnp.sum(t1_ref[...] * v1, axis=1, keepdims=True) + off
    s2_ref[...] = jnp.sum(t2_ref[...] * (v1 + v2), axis=1, keepdims=True) + off


def _project_tables(table_1, table_2, w1a, w1b, w2a, w2b, b1a, b1b, b2a, b2b):
    grid = (VOCAB // ROWS_PER_STEP,)
    full = lambda shape: pl.BlockSpec(shape, lambda i: (0, 0))
    s1, s2 = pl.pallas_call(
        _project_body,
        grid=grid,
        in_specs=[
            pl.BlockSpec((ROWS_PER_STEP, EMB_DIM), lambda i: (i, 0)),
            pl.BlockSpec((ROWS_PER_STEP, EMB_DIM), lambda i: (i, 0)),
            full((512, EMB_DIM)),
            full((1, 512)),
            full((512, EMB_DIM)),
            full((1, 512)),
            full((1, 512)),
            full((1, 1)),
            full((1, 512)),
            full((1, 1)),
        ],
        out_specs=[
            pl.BlockSpec((ROWS_PER_STEP, 1), lambda i: (i, 0)),
            pl.BlockSpec((ROWS_PER_STEP, 1), lambda i: (i, 0)),
        ],
        out_shape=[
            jax.ShapeDtypeStruct((VOCAB, 1), jnp.float32),
            jax.ShapeDtypeStruct((VOCAB, 1), jnp.float32),
        ],
    )(table_1, table_2, w1a.T, w1b.T, w2a.T, w2b.T,
      b1a.reshape(1, 512), b1b.reshape(1, 1),
      b2a.reshape(1, 512), b2b.reshape(1, 1))
    return s1.reshape(VOCAB), s2.reshape(VOCAB)


NUM_CORES = 2
NUM_SUBCORES = 16
NUM_WORKERS = NUM_CORES * NUM_SUBCORES
ROWS_PER_WORKER = BATCH // NUM_WORKERS


def _sc_body(s1_hbm, s2_hbm, xt_hbm, yt_hbm, out_hbm,
             idx1_v, idx2_v, vals1_v, vals2_v, out_v, sem):
    wid = lax.axis_index("s") * NUM_CORES + lax.axis_index("c")
    base = wid * ROWS_PER_WORKER

    pltpu.sync_copy(xt_hbm.at[wid], idx1_v)
    pltpu.sync_copy(yt_hbm.at[wid], idx2_v)
    descs = []
    for j in range(BAG):
        descs.append(pltpu.async_copy(s1_hbm.at[idx1_v.at[j]], vals1_v.at[j], sem))
        descs.append(pltpu.async_copy(s2_hbm.at[idx2_v.at[j]], vals2_v.at[j], sem))
    for d in descs:
        d.wait()

    for k in range(ROWS_PER_WORKER // 16):
        sl = pl.ds(k * 16, 16)
        acc = vals1_v[0, sl] + vals2_v[0, sl]
        for j in range(1, BAG):
            acc = acc + vals1_v[j, sl]
            acc = acc + vals2_v[j, sl]
        out_v[sl] = acc

    pltpu.sync_copy(out_v, out_hbm.at[pl.ds(base, ROWS_PER_WORKER)])


@functools.lru_cache(maxsize=None)
def _sc_bag_sum():
    return functools.partial(
        pl.kernel,
        mesh=plsc.VectorSubcoreMesh(core_axis_name="c", subcore_axis_name="s"),
        out_type=jax.ShapeDtypeStruct((BATCH,), jnp.float32),
        scratch_types=[
            pltpu.VMEM((BAG, ROWS_PER_WORKER), jnp.int32),
            pltpu.VMEM((BAG, ROWS_PER_WORKER), jnp.int32),
            pltpu.VMEM((BAG, ROWS_PER_WORKER), jnp.float32),
            pltpu.VMEM((BAG, ROWS_PER_WORKER), jnp.float32),
            pltpu.VMEM((ROWS_PER_WORKER,), jnp.float32),
            pltpu.SemaphoreType.DMA,
        ],
    )(_sc_body)


def kernel(x, y, table_1, table_2, w1a, b1a, w1b, b1b, w2a, b2a, w2b, b2b):
    s1, s2 = _project_tables(table_1, table_2, w1a, w1b, w2a, w2b,
                             b1a, b1b, b2a, b2b)
    xt = x.reshape(NUM_WORKERS, ROWS_PER_WORKER, BAG).transpose(0, 2, 1)
    yt = y.reshape(NUM_WORKERS, ROWS_PER_WORKER, BAG).transpose(0, 2, 1)
    out = _sc_bag_sum()(s1, s2, xt, yt)
    return out.reshape(BATCH, 1)
